# baseline (speedup 1.0000x reference)
import jax
import jax.numpy as jnp
from jax.experimental import pallas as pl
from jax.experimental.pallas import tpu as pltpu


def _c2q_body(u_ref, s_ref, o_ref, a_buf, r_buf):
    i = pl.program_id(0)
    n = pl.num_programs(0) - 1
    slot = jax.lax.rem(i, 2)
    prev = jax.lax.rem(i + 1, 2)

    s = s_ref[...]
    m = jnp.max(s, axis=1, keepdims=True)
    e = jnp.exp(s - m)
    denom = jnp.sum(e, axis=1)
    r_buf[slot] = (1.0 / denom).reshape(1, -1)
    a = e.astype(jnp.bfloat16)
    half = a.shape[1] // 2
    a_buf[slot, :, :half] = a[:, :half]
    a_buf[slot, :, half:] = a[:, half:]

    out = jax.lax.dot_general(
        u_ref[...], a_buf[prev],
        dimension_numbers=(((0,), (1,)), ((), ())),
        preferred_element_type=jnp.float32,
    )
    o_ref[...] = out * r_buf[prev]


def kernel(u, s):
    t, j = s.shape
    d = u.shape[2]
    ub = u[0].astype(jnp.bfloat16)
    bt = 512
    n = t // bt
    return pl.pallas_call(
        _c2q_body,
        grid=(n + 1,),
        in_specs=[
            pl.BlockSpec((j, d), lambda i: (0, 0)),
            pl.BlockSpec((bt, j), lambda i: (jnp.minimum(i, n - 1), 0)),
        ],
        out_specs=pl.BlockSpec((d, bt), lambda i: (0, jnp.maximum(i - 1, 0))),
        out_shape=jax.ShapeDtypeStruct((d, t), jnp.float32),
        scratch_shapes=[
            pltpu.VMEM((2, bt, j), jnp.bfloat16),
            pltpu.VMEM((2, 1, bt), jnp.float32),
        ],
        compiler_params=pltpu.CompilerParams(
            dimension_semantics=("arbitrary",),
            vmem_limit_bytes=56 * 1024 * 1024,
        ),
        name="context2query_fused",
    )(ub, s)
